# baseline (speedup 1.0000x reference)
import functools

import jax
import jax.numpy as jnp
from jax import lax
from jax.experimental import pallas as pl
from jax.experimental.pallas import tpu as pltpu
from jax.experimental.pallas import tpu_sc as plsc

_ROW = 1
_D = 64
_L = 16

_mesh = plsc.VectorSubcoreMesh(
    core_axis_name="c", subcore_axis_name="s", num_cores=1, num_subcores=1
)


@functools.partial(
    pl.kernel,
    mesh=_mesh,
    out_type=jax.ShapeDtypeStruct((2, 2, 1, _D), jnp.float32),
    scratch_types=[
        pltpu.VMEM((_D, 128), jnp.float32),
        pltpu.VMEM((1, _D), jnp.float32),
    ],
    compiler_params=pltpu.CompilerParams(
        use_tc_tiling_on_sc=True, needs_layout_passes=False
    ),
)
def _gather_row(xt_hbm, out_hbm, tile_v, row_v):
    is_w0 = (lax.axis_index("s") == 0) & (lax.axis_index("c") == 0)

    @pl.when(is_w0)
    def _():
        pltpu.sync_copy(xt_hbm.at[pl.ds(0, _D), pl.ds(0, 128)], tile_v)
        col = jnp.full((_L,), _ROW, dtype=jnp.int32)
        for c in range(_D // _L):
            rows = jnp.arange(_L, dtype=jnp.int32) + c * _L
            row_v[0, pl.ds(c * _L, _L)] = plsc.load_gather(tile_v, [rows, col])
        for i in range(2):
            for j in range(2):
                pltpu.sync_copy(row_v, out_hbm.at[i, j])


def kernel(x):
    return _gather_row(x.T)
